# Initial kernel scaffold; baseline (speedup 1.0000x reference)
import jax
import jax.numpy as jnp
from jax.experimental import pallas as pl


def kernel(user_emb, item_emb, edge_index, edge_vals, W1_0, b1_0, W2_0, b2_0, W1_1, b1_1, W2_1, b2_1, W1_2, b1_2, W2_2, b2_2, users, pos_items, neg_items, node_flag):
    raise NotImplementedError("write your pallas kernel here")


# speedup vs baseline: 5.4561x; 5.4561x over previous
import functools

import jax
import jax.numpy as jnp
from jax import lax
from jax.experimental import pallas as pl
from jax.experimental.pallas import tpu as pltpu
from jax.experimental.pallas import tpu_sc as plsc

N_USER = 25000
N_ITEM = 25000
N_NODES = N_USER + N_ITEM
EMB = 64
HALF = 32
N_EDGES = 800000
BATCH = 4096

GROUP = 128
N_GROUPS = N_EDGES // GROUP
BASE_GROUPS = N_GROUPS // 16
EXTRA_TILES = N_GROUPS % 16
CH_GROUPS = 26
CH_EDGES = CH_GROUPS * GROUP
N_CHUNKS = BASE_GROUPS // CH_GROUPS
RB_ROWS = 3128
RB_LAST = N_NODES - 15 * RB_ROWS
ZROWS = 125

_mesh = plsc.VectorSubcoreMesh(core_axis_name="c", subcore_axis_name="s")


@functools.partial(
    pl.kernel,
    out_type=jax.ShapeDtypeStruct((2 * N_NODES, HALF), jnp.float32),
    mesh=_mesh,
    scratch_types=[
        pltpu.VMEM((CH_EDGES,), jnp.int32),
        pltpu.VMEM((CH_EDGES,), jnp.int32),
        pltpu.VMEM((CH_EDGES,), jnp.float32),
        pltpu.VMEM((GROUP,), jnp.int32),
        pltpu.VMEM((GROUP,), jnp.int32),
        pltpu.VMEM((GROUP, HALF), jnp.float32),
        pltpu.VMEM((ZROWS, HALF), jnp.float32),
        pltpu.VMEM_SHARED((N_NODES, HALF), jnp.float32),
        pltpu.SemaphoreType.DMA,
    ],
    compiler_params=pltpu.CompilerParams(use_tc_tiling_on_sc=False),
)
def _sc_spmm(e_hbm, src_hbm, dst_hbm, vals_hbm, out_hbm,
             src_m, dst_m, vals_m, src_g, dst_g, rows_v, zero_v, acc, sem):
    c = lax.axis_index("c")
    s = lax.axis_index("s")
    coff = c * N_NODES

    zvec = jnp.zeros((16,), jnp.float32)

    def zfill(r, carry):
        zero_v[r, pl.ds(0, 16)] = zvec
        zero_v[r, pl.ds(16, 16)] = zvec
        return carry

    lax.fori_loop(0, ZROWS, zfill, 0)

    def zcopy(i, carry):
        pltpu.sync_copy(zero_v,
                        acc.at[pl.ds(s * (N_NODES // 16) + i * ZROWS, ZROWS)])
        return carry

    lax.fori_loop(0, (N_NODES // 16) // ZROWS, zcopy, 0)
    plsc.subcore_barrier()

    g0 = s * BASE_GROUPS + jnp.minimum(s, EXTRA_TILES)

    def do_group(goff):
        for k in range(GROUP // 16):
            sl = pl.ds(k * 16, 16)
            src_g[sl] = src_m[pl.ds(goff + k * 16, 16)] + coff
            dst_g[sl] = dst_m[pl.ds(goff + k * 16, 16)]
        pltpu.async_copy(e_hbm.at[src_g], rows_v, sem).wait()

        def scale(g, carry):
            vv = vals_m[pl.ds(goff + g * 16, 16)]
            for jj in range(16):
                e = g * 16 + jj
                vsp = jnp.full((16,), vv[jj], jnp.float32)
                lo = pl.ds(0, 16)
                hi = pl.ds(16, 16)
                rows_v[e, lo] = rows_v[e, lo] * vsp
                rows_v[e, hi] = rows_v[e, hi] * vsp
            return carry

        lax.fori_loop(0, GROUP // 16, scale, 0)
        pltpu.sync_copy(rows_v, acc.at[dst_g], add=True)

    def chunk_body(i, carry):
        eb = (g0 + i * CH_GROUPS) * GROUP
        pltpu.sync_copy(src_hbm.at[pl.ds(eb, CH_EDGES)], src_m)
        pltpu.sync_copy(dst_hbm.at[pl.ds(eb, CH_EDGES)], dst_m)
        pltpu.sync_copy(vals_hbm.at[pl.ds(eb, CH_EDGES)], vals_m)

        def group_body(j, carry2):
            do_group(j * GROUP)
            return carry2

        lax.fori_loop(0, CH_GROUPS, group_body, 0)
        return carry

    lax.fori_loop(0, N_CHUNKS, chunk_body, 0)

    @pl.when(s < EXTRA_TILES)
    def _tail():
        eb = (g0 + BASE_GROUPS) * GROUP
        pltpu.sync_copy(src_hbm.at[pl.ds(eb, GROUP)],
                        src_m.at[pl.ds(0, GROUP)])
        pltpu.sync_copy(dst_hbm.at[pl.ds(eb, GROUP)],
                        dst_m.at[pl.ds(0, GROUP)])
        pltpu.sync_copy(vals_hbm.at[pl.ds(eb, GROUP)],
                        vals_m.at[pl.ds(0, GROUP)])
        do_group(0)

    plsc.subcore_barrier()

    @pl.when(s < 15)
    def _rb_main():
        rb = s * RB_ROWS
        pltpu.sync_copy(acc.at[pl.ds(rb, RB_ROWS)],
                        out_hbm.at[pl.ds(coff + rb, RB_ROWS)])

    @pl.when(s == 15)
    def _rb_last():
        rb = 15 * RB_ROWS
        pltpu.sync_copy(acc.at[pl.ds(rb, RB_LAST)],
                        out_hbm.at[pl.ds(coff + rb, RB_LAST)])


_DENSE_BLOCK = 2000


def _dense_body(l_ref, e_ref, w1_ref, b1_ref, w2_ref, b2_ref,
                enorm_ref, eo_ref):
    L = jnp.concatenate([l_ref[0], l_ref[1]], axis=1)
    E = jnp.concatenate([e_ref[0], e_ref[1]], axis=1)
    H = (jnp.dot(L + E, w1_ref[...], preferred_element_type=jnp.float32)
         + jnp.dot(L * E, w2_ref[...], preferred_element_type=jnp.float32)
         + b1_ref[0] + b2_ref[0])
    Eo = jnp.where(H >= 0, H, 0.2 * H)
    nrm = jnp.sqrt(jnp.sum(Eo * Eo, axis=1, keepdims=True))
    enorm_ref[...] = Eo / jnp.maximum(nrm, 1e-12)
    eo_ref[0] = Eo[:, :HALF]
    eo_ref[1] = Eo[:, HALF:]


_dense_tc = pl.pallas_call(
    _dense_body,
    grid=(N_NODES // _DENSE_BLOCK,),
    in_specs=[
        pl.BlockSpec((2, _DENSE_BLOCK, HALF), lambda i: (0, i, 0)),
        pl.BlockSpec((2, _DENSE_BLOCK, HALF), lambda i: (0, i, 0)),
        pl.BlockSpec((EMB, EMB), lambda i: (0, 0)),
        pl.BlockSpec((1, EMB), lambda i: (0, 0)),
        pl.BlockSpec((EMB, EMB), lambda i: (0, 0)),
        pl.BlockSpec((1, EMB), lambda i: (0, 0)),
    ],
    out_specs=[
        pl.BlockSpec((_DENSE_BLOCK, EMB), lambda i: (i, 0)),
        pl.BlockSpec((2, _DENSE_BLOCK, HALF), lambda i: (0, i, 0)),
    ],
    out_shape=[
        jax.ShapeDtypeStruct((N_NODES, EMB), jnp.float32),
        jax.ShapeDtypeStruct((2, N_NODES, HALF), jnp.float32),
    ],
)

GCHUNK = BATCH // 32


@functools.partial(
    pl.kernel,
    out_type=[jax.ShapeDtypeStruct((BATCH, 4 * EMB), jnp.float32)
              for _ in range(3)],
    mesh=_mesh,
    scratch_types=[
        pltpu.VMEM((GCHUNK,), jnp.int32),
        pltpu.VMEM((GCHUNK, EMB), jnp.float32),
        pltpu.SemaphoreType.DMA,
    ],
    compiler_params=pltpu.CompilerParams(use_tc_tiling_on_sc=False),
)
def _sc_lookup(t0, t1, t2, t3, users, pos, neg,
               u_out, p_out, n_out, idx_v, buf, sem):
    c = lax.axis_index("c")
    s = lax.axis_index("s")
    w = s * 2 + c
    r0 = w * GCHUNK
    for idx_hbm, out_hbm, off in ((users, u_out, -1),
                                  (pos, p_out, N_USER - 1),
                                  (neg, n_out, N_USER - 1)):
        pltpu.sync_copy(idx_hbm.at[pl.ds(r0, GCHUNK)], idx_v)
        for k in range(GCHUNK // 16):
            sl = pl.ds(k * 16, 16)
            idx_v[sl] = idx_v[sl] + off
        for k, tbl in enumerate((t0, t1, t2, t3)):
            pltpu.async_copy(tbl.at[idx_v], buf, sem).wait()
            pltpu.sync_copy(buf, out_hbm.at[pl.ds(r0, GCHUNK),
                                            pl.ds(k * EMB, EMB)])


def kernel(user_emb, item_emb, edge_index, edge_vals,
           W1_0, b1_0, W2_0, b2_0, W1_1, b1_1, W2_1, b2_1,
           W1_2, b1_2, W2_2, b2_2,
           users, pos_items, neg_items, node_flag):
    Ws = [(W1_0, b1_0, W2_0, b2_0), (W1_1, b1_1, W2_1, b2_1),
          (W1_2, b1_2, W2_2, b2_2)]
    E0 = jnp.concatenate([user_emb, item_emb], axis=0)
    estack = jnp.stack([E0[:, :HALF], E0[:, HALF:]], axis=0)
    src = edge_index[0]
    dst = edge_index[1]

    norms = []
    for (W1, b1, W2, b2) in Ws:
        lflat = _sc_spmm(estack.reshape(2 * N_NODES, HALF),
                         src, dst, edge_vals)
        enorm, estack = _dense_tc(lflat.reshape(2, N_NODES, HALF), estack,
                                  W1, b1.reshape(1, EMB),
                                  W2, b2.reshape(1, EMB))
        norms.append(enorm)

    u, p, n = _sc_lookup(E0, norms[0], norms[1], norms[2],
                         users, pos_items, neg_items)
    return (u, p, n)


# speedup vs baseline: 8.3218x; 1.5252x over previous
import functools

import jax
import jax.numpy as jnp
from jax import lax
from jax.experimental import pallas as pl
from jax.experimental.pallas import tpu as pltpu
from jax.experimental.pallas import tpu_sc as plsc

N_USER = 25000
N_ITEM = 25000
N_NODES = N_USER + N_ITEM
EMB = 64
HALF = 32
N_EDGES = 800000
BATCH = 4096

GROUP = 128
N_GROUPS = N_EDGES // GROUP
BASE_GROUPS = N_GROUPS // 16
EXTRA_TILES = N_GROUPS % 16
CH_GROUPS = 26
CH_EDGES = CH_GROUPS * GROUP
N_CHUNKS = BASE_GROUPS // CH_GROUPS
RB_ROWS = 3128
RB_LAST = N_NODES - 15 * RB_ROWS
ZROWS = 125

_mesh = plsc.VectorSubcoreMesh(core_axis_name="c", subcore_axis_name="s")


NSLOT = 3


@functools.partial(
    pl.kernel,
    out_type=jax.ShapeDtypeStruct((2 * N_NODES, HALF), jnp.float32),
    mesh=_mesh,
    scratch_types=[
        pltpu.VMEM((CH_EDGES,), jnp.int32),
        pltpu.VMEM((CH_EDGES,), jnp.int32),
        pltpu.VMEM((CH_EDGES,), jnp.float32),
        pltpu.VMEM((NSLOT, GROUP), jnp.int32),
        pltpu.VMEM((NSLOT, GROUP, HALF), jnp.float32),
        pltpu.VMEM_SHARED((N_NODES, HALF), jnp.float32),
        pltpu.SemaphoreType.DMA((NSLOT,)),
        pltpu.SemaphoreType.DMA((NSLOT,)),
    ],
    compiler_params=pltpu.CompilerParams(use_tc_tiling_on_sc=False),
)
def _sc_spmm(e_hbm, src_hbm, dst_hbm, vals_hbm, zeros_hbm, out_hbm,
             src_m, dst_m, vals_m, dst_g, rows_v, acc, gsem, ssem):
    c = lax.axis_index("c")
    s = lax.axis_index("s")
    coff = c * N_NODES

    zr = s * (N_NODES // 16)
    pltpu.sync_copy(zeros_hbm.at[pl.ds(zr, N_NODES // 16)],
                    acc.at[pl.ds(zr, N_NODES // 16)])
    plsc.subcore_barrier()

    g0 = s * BASE_GROUPS + jnp.minimum(s, EXTRA_TILES)
    splat_idx = [jnp.full((16, 1), jj, jnp.int32) for jj in range(16)]
    gdn = lax.GatherDimensionNumbers(offset_dims=(), collapsed_slice_dims=(0,),
                                     start_index_map=(0,))

    def scale_group(b, base):
        def sbody(g, carry):
            vv = vals_m[pl.ds(base + g * 16, 16)]
            for jj in range(16):
                e = g * 16 + jj
                vsp = lax.gather(vv, splat_idx[jj], gdn, (1,),
                                 mode=lax.GatherScatterMode.PROMISE_IN_BOUNDS)
                lo = pl.ds(0, 16)
                hi = pl.ds(16, 16)
                rows_v[b, e, lo] = rows_v[b, e, lo] * vsp
                rows_v[b, e, hi] = rows_v[b, e, hi] * vsp
            return carry

        lax.fori_loop(0, GROUP // 16, sbody, 0)

    def copy_dst(j, b):
        for k in range(GROUP // 16):
            dst_g[b, pl.ds(k * 16, 16)] = dst_m[pl.ds(j * GROUP + k * 16, 16)]

    def start_gather(j, b):
        return pltpu.async_copy(
            e_hbm.at[src_m.at[pl.ds(j * GROUP, GROUP)]],
            rows_v.at[b], gsem.at[b])

    def start_scatter(b):
        return pltpu.async_copy(rows_v.at[b], acc.at[dst_g.at[b]],
                                ssem.at[b], add=True)

    def chunk_body(i, carry):
        eb = (g0 + i * CH_GROUPS) * GROUP
        pltpu.sync_copy(src_hbm.at[pl.ds(eb, CH_EDGES)], src_m)
        pltpu.sync_copy(dst_hbm.at[pl.ds(eb, CH_EDGES)], dst_m)
        pltpu.sync_copy(vals_hbm.at[pl.ds(eb, CH_EDGES)], vals_m)

        def offs(k, carry2):
            sl = pl.ds(k * 16, 16)
            src_m[sl] = src_m[sl] + coff
            return carry2

        lax.fori_loop(0, CH_EDGES // 16, offs, 0)

        scat = {}
        prev = None
        for j in range(CH_GROUPS):
            b = j % NSLOT
            if j >= NSLOT:
                scat.pop(j - NSLOT).wait()
            copy_dst(j, b)
            gat = start_gather(j, b)
            if prev is not None:
                pj, pb, pgat = prev
                pgat.wait()
                scale_group(pb, pj * GROUP)
                scat[pj] = start_scatter(pb)
            prev = (j, b, gat)
        pj, pb, pgat = prev
        pgat.wait()
        scale_group(pb, pj * GROUP)
        scat[pj] = start_scatter(pb)
        for j in sorted(scat):
            scat.pop(j).wait()
        return carry

    lax.fori_loop(0, N_CHUNKS, chunk_body, 0)

    @pl.when(s < EXTRA_TILES)
    def _tail():
        eb = (g0 + BASE_GROUPS) * GROUP
        pltpu.sync_copy(src_hbm.at[pl.ds(eb, GROUP)],
                        src_m.at[pl.ds(0, GROUP)])
        pltpu.sync_copy(dst_hbm.at[pl.ds(eb, GROUP)],
                        dst_m.at[pl.ds(0, GROUP)])
        pltpu.sync_copy(vals_hbm.at[pl.ds(eb, GROUP)],
                        vals_m.at[pl.ds(0, GROUP)])

        def offs(k, carry2):
            sl = pl.ds(k * 16, 16)
            src_m[sl] = src_m[sl] + coff
            return carry2

        lax.fori_loop(0, GROUP // 16, offs, 0)
        copy_dst(0, 0)
        start_gather(0, 0).wait()
        scale_group(0, 0)
        start_scatter(0).wait()

    plsc.subcore_barrier()

    @pl.when(s < 15)
    def _rb_main():
        rb = s * RB_ROWS
        pltpu.sync_copy(acc.at[pl.ds(rb, RB_ROWS)],
                        out_hbm.at[pl.ds(coff + rb, RB_ROWS)])

    @pl.when(s == 15)
    def _rb_last():
        rb = 15 * RB_ROWS
        pltpu.sync_copy(acc.at[pl.ds(rb, RB_LAST)],
                        out_hbm.at[pl.ds(coff + rb, RB_LAST)])


_DENSE_BLOCK = 2000


def _dense_body(l_ref, e_ref, w1_ref, b1_ref, w2_ref, b2_ref,
                enorm_ref, eo_ref):
    L = jnp.concatenate([l_ref[0], l_ref[1]], axis=1)
    E = jnp.concatenate([e_ref[0], e_ref[1]], axis=1)
    H = (jnp.dot(L + E, w1_ref[...], preferred_element_type=jnp.float32)
         + jnp.dot(L * E, w2_ref[...], preferred_element_type=jnp.float32)
         + b1_ref[0] + b2_ref[0])
    Eo = jnp.where(H >= 0, H, 0.2 * H)
    nrm = jnp.sqrt(jnp.sum(Eo * Eo, axis=1, keepdims=True))
    enorm_ref[...] = Eo / jnp.maximum(nrm, 1e-12)
    eo_ref[0] = Eo[:, :HALF]
    eo_ref[1] = Eo[:, HALF:]


_dense_tc = pl.pallas_call(
    _dense_body,
    grid=(N_NODES // _DENSE_BLOCK,),
    in_specs=[
        pl.BlockSpec((2, _DENSE_BLOCK, HALF), lambda i: (0, i, 0)),
        pl.BlockSpec((2, _DENSE_BLOCK, HALF), lambda i: (0, i, 0)),
        pl.BlockSpec((EMB, EMB), lambda i: (0, 0)),
        pl.BlockSpec((1, EMB), lambda i: (0, 0)),
        pl.BlockSpec((EMB, EMB), lambda i: (0, 0)),
        pl.BlockSpec((1, EMB), lambda i: (0, 0)),
    ],
    out_specs=[
        pl.BlockSpec((_DENSE_BLOCK, EMB), lambda i: (i, 0)),
        pl.BlockSpec((2, _DENSE_BLOCK, HALF), lambda i: (0, i, 0)),
    ],
    out_shape=[
        jax.ShapeDtypeStruct((N_NODES, EMB), jnp.float32),
        jax.ShapeDtypeStruct((2, N_NODES, HALF), jnp.float32),
    ],
)

GCHUNK = BATCH // 32


@functools.partial(
    pl.kernel,
    out_type=[jax.ShapeDtypeStruct((BATCH, 4 * EMB), jnp.float32)
              for _ in range(3)],
    mesh=_mesh,
    scratch_types=[
        pltpu.VMEM((GCHUNK,), jnp.int32),
        pltpu.VMEM((GCHUNK, EMB), jnp.float32),
        pltpu.SemaphoreType.DMA,
    ],
    compiler_params=pltpu.CompilerParams(use_tc_tiling_on_sc=False),
)
def _sc_lookup(t0, t1, t2, t3, users, pos, neg,
               u_out, p_out, n_out, idx_v, buf, sem):
    c = lax.axis_index("c")
    s = lax.axis_index("s")
    w = s * 2 + c
    r0 = w * GCHUNK
    for idx_hbm, out_hbm, off in ((users, u_out, -1),
                                  (pos, p_out, N_USER - 1),
                                  (neg, n_out, N_USER - 1)):
        pltpu.sync_copy(idx_hbm.at[pl.ds(r0, GCHUNK)], idx_v)
        for k in range(GCHUNK // 16):
            sl = pl.ds(k * 16, 16)
            idx_v[sl] = idx_v[sl] + off
        for k, tbl in enumerate((t0, t1, t2, t3)):
            pltpu.async_copy(tbl.at[idx_v], buf, sem).wait()
            pltpu.sync_copy(buf, out_hbm.at[pl.ds(r0, GCHUNK),
                                            pl.ds(k * EMB, EMB)])


def kernel(user_emb, item_emb, edge_index, edge_vals,
           W1_0, b1_0, W2_0, b2_0, W1_1, b1_1, W2_1, b2_1,
           W1_2, b1_2, W2_2, b2_2,
           users, pos_items, neg_items, node_flag):
    Ws = [(W1_0, b1_0, W2_0, b2_0), (W1_1, b1_1, W2_1, b2_1),
          (W1_2, b1_2, W2_2, b2_2)]
    E0 = jnp.concatenate([user_emb, item_emb], axis=0)
    estack = jnp.stack([E0[:, :HALF], E0[:, HALF:]], axis=0)
    src = edge_index[0]
    dst = edge_index[1]
    zeros = jnp.zeros((N_NODES, HALF), jnp.float32)

    norms = []
    for (W1, b1, W2, b2) in Ws:
        lflat = _sc_spmm(estack.reshape(2 * N_NODES, HALF),
                         src, dst, edge_vals, zeros)
        enorm, estack = _dense_tc(lflat.reshape(2, N_NODES, HALF), estack,
                                  W1, b1.reshape(1, EMB),
                                  W2, b2.reshape(1, EMB))
        norms.append(enorm)

    u, p, n = _sc_lookup(E0, norms[0], norms[1], norms[2],
                         users, pos_items, neg_items)
    return (u, p, n)


# speedup vs baseline: 8.5614x; 1.0288x over previous
import functools

import jax
import jax.numpy as jnp
from jax import lax
from jax.experimental import pallas as pl
from jax.experimental.pallas import tpu as pltpu
from jax.experimental.pallas import tpu_sc as plsc

N_USER = 25000
N_ITEM = 25000
N_NODES = N_USER + N_ITEM
EMB = 64
HALF = 32
N_EDGES = 800000
BATCH = 4096

GROUP = 128
N_GROUPS = N_EDGES // GROUP
BASE_GROUPS = N_GROUPS // 16
EXTRA_TILES = N_GROUPS % 16
CH_GROUPS = 13
CH_EDGES = CH_GROUPS * GROUP
N_CHUNKS = BASE_GROUPS // CH_GROUPS
RB_ROWS = 3128
RB_LAST = N_NODES - 15 * RB_ROWS
ZROWS = 125

_mesh = plsc.VectorSubcoreMesh(core_axis_name="c", subcore_axis_name="s")


NSLOT = 3


@functools.partial(
    pl.kernel,
    out_type=jax.ShapeDtypeStruct((2 * N_NODES, HALF), jnp.float32),
    mesh=_mesh,
    scratch_types=[
        pltpu.VMEM((2, CH_EDGES), jnp.int32),
        pltpu.VMEM((2, CH_EDGES), jnp.int32),
        pltpu.VMEM((2, CH_EDGES), jnp.float32),
        pltpu.VMEM((NSLOT, GROUP), jnp.int32),
        pltpu.VMEM((NSLOT, GROUP, HALF), jnp.float32),
        pltpu.VMEM_SHARED((N_NODES, HALF), jnp.float32),
        pltpu.SemaphoreType.DMA((NSLOT,)),
        pltpu.SemaphoreType.DMA((NSLOT,)),
        pltpu.SemaphoreType.DMA,
    ],
    compiler_params=pltpu.CompilerParams(use_tc_tiling_on_sc=False),
)
def _sc_spmm(e_hbm, src_hbm, dst_hbm, vals_hbm, zeros_hbm, out_hbm,
             src_m, dst_m, vals_m, dst_g, rows_v, acc, gsem, ssem, stsem):
    c = lax.axis_index("c")
    s = lax.axis_index("s")
    coff = c * N_NODES

    zr = s * (N_NODES // 16)
    pltpu.sync_copy(zeros_hbm.at[pl.ds(zr, N_NODES // 16)],
                    acc.at[pl.ds(zr, N_NODES // 16)])
    plsc.subcore_barrier()

    g0 = s * BASE_GROUPS + jnp.minimum(s, EXTRA_TILES)
    splat_idx = [jnp.full((16, 1), jj, jnp.int32) for jj in range(16)]
    gdn = lax.GatherDimensionNumbers(offset_dims=(), collapsed_slice_dims=(0,),
                                     start_index_map=(0,))

    def scale_group(pb, b, base):
        def sbody(g, carry):
            vv = vals_m[pb, pl.ds(base + g * 16, 16)]
            for jj in range(16):
                e = g * 16 + jj
                vsp = lax.gather(vv, splat_idx[jj], gdn, (1,),
                                 mode=lax.GatherScatterMode.PROMISE_IN_BOUNDS)
                lo = pl.ds(0, 16)
                hi = pl.ds(16, 16)
                rows_v[b, e, lo] = rows_v[b, e, lo] * vsp
                rows_v[b, e, hi] = rows_v[b, e, hi] * vsp
            return carry

        lax.fori_loop(0, GROUP // 16, sbody, 0)

    def copy_dst(pb, j, b):
        for k in range(GROUP // 16):
            dst_g[b, pl.ds(k * 16, 16)] = dst_m[pb, pl.ds(j * GROUP + k * 16, 16)]

    def start_gather(pb, j, b):
        return pltpu.async_copy(
            e_hbm.at[src_m.at[pb, pl.ds(j * GROUP, GROUP)]],
            rows_v.at[b], gsem.at[b])

    def start_scatter(b):
        return pltpu.async_copy(rows_v.at[b], acc.at[dst_g.at[b]],
                                ssem.at[b], add=True)

    def stage(i, pb):
        eb = (g0 + i * CH_GROUPS) * GROUP
        return (pltpu.make_async_copy(src_hbm.at[pl.ds(eb, CH_EDGES)],
                                      src_m.at[pb], stsem),
                pltpu.make_async_copy(dst_hbm.at[pl.ds(eb, CH_EDGES)],
                                      dst_m.at[pb], stsem),
                pltpu.make_async_copy(vals_hbm.at[pl.ds(eb, CH_EDGES)],
                                      vals_m.at[pb], stsem))

    for cp in stage(0, 0):
        cp.start()

    def chunk_body(i, carry):
        pb = lax.rem(i, 2)
        for cp in stage(i, pb):
            cp.wait()

        @pl.when(i < N_CHUNKS - 1)
        def _prefetch():
            for cp in stage(i + 1, 1 - pb):
                cp.start()

        def offs(k, carry2):
            sl = pl.ds(k * 16, 16)
            src_m[pb, sl] = src_m[pb, sl] + coff
            return carry2

        lax.fori_loop(0, CH_EDGES // 16, offs, 0)

        scat = {}
        prev = None
        for j in range(CH_GROUPS):
            b = j % NSLOT
            if j >= NSLOT:
                scat.pop(j - NSLOT).wait()
            copy_dst(pb, j, b)
            gat = start_gather(pb, j, b)
            if prev is not None:
                pj, pbuf, pgat = prev
                pgat.wait()
                scale_group(pb, pbuf, pj * GROUP)
                scat[pj] = start_scatter(pbuf)
            prev = (j, b, gat)
        pj, pbuf, pgat = prev
        pgat.wait()
        scale_group(pb, pbuf, pj * GROUP)
        scat[pj] = start_scatter(pbuf)
        for j in sorted(scat):
            scat.pop(j).wait()
        return carry

    lax.fori_loop(0, N_CHUNKS, chunk_body, 0)

    @pl.when(s < EXTRA_TILES)
    def _tail():
        eb = (g0 + BASE_GROUPS) * GROUP
        pltpu.sync_copy(src_hbm.at[pl.ds(eb, GROUP)],
                        src_m.at[0, pl.ds(0, GROUP)])
        pltpu.sync_copy(dst_hbm.at[pl.ds(eb, GROUP)],
                        dst_m.at[0, pl.ds(0, GROUP)])
        pltpu.sync_copy(vals_hbm.at[pl.ds(eb, GROUP)],
                        vals_m.at[0, pl.ds(0, GROUP)])

        def offs(k, carry2):
            sl = pl.ds(k * 16, 16)
            src_m[0, sl] = src_m[0, sl] + coff
            return carry2

        lax.fori_loop(0, GROUP // 16, offs, 0)
        copy_dst(0, 0, 0)
        start_gather(0, 0, 0).wait()
        scale_group(0, 0, 0)
        start_scatter(0).wait()

    plsc.subcore_barrier()

    @pl.when(s < 15)
    def _rb_main():
        rb = s * RB_ROWS
        pltpu.sync_copy(acc.at[pl.ds(rb, RB_ROWS)],
                        out_hbm.at[pl.ds(coff + rb, RB_ROWS)])

    @pl.when(s == 15)
    def _rb_last():
        rb = 15 * RB_ROWS
        pltpu.sync_copy(acc.at[pl.ds(rb, RB_LAST)],
                        out_hbm.at[pl.ds(coff + rb, RB_LAST)])


_DENSE_BLOCK = 2000


def _dense_body(l_ref, e_ref, w1_ref, b1_ref, w2_ref, b2_ref,
                enorm_ref, eo_ref):
    L = jnp.concatenate([l_ref[0], l_ref[1]], axis=1)
    E = jnp.concatenate([e_ref[0], e_ref[1]], axis=1)
    H = (jnp.dot(L + E, w1_ref[...], preferred_element_type=jnp.float32)
         + jnp.dot(L * E, w2_ref[...], preferred_element_type=jnp.float32)
         + b1_ref[0] + b2_ref[0])
    Eo = jnp.where(H >= 0, H, 0.2 * H)
    nrm = jnp.sqrt(jnp.sum(Eo * Eo, axis=1, keepdims=True))
    enorm_ref[...] = Eo / jnp.maximum(nrm, 1e-12)
    eo_ref[0] = Eo[:, :HALF]
    eo_ref[1] = Eo[:, HALF:]


_dense_tc = pl.pallas_call(
    _dense_body,
    grid=(N_NODES // _DENSE_BLOCK,),
    in_specs=[
        pl.BlockSpec((2, _DENSE_BLOCK, HALF), lambda i: (0, i, 0)),
        pl.BlockSpec((2, _DENSE_BLOCK, HALF), lambda i: (0, i, 0)),
        pl.BlockSpec((EMB, EMB), lambda i: (0, 0)),
        pl.BlockSpec((1, EMB), lambda i: (0, 0)),
        pl.BlockSpec((EMB, EMB), lambda i: (0, 0)),
        pl.BlockSpec((1, EMB), lambda i: (0, 0)),
    ],
    out_specs=[
        pl.BlockSpec((_DENSE_BLOCK, EMB), lambda i: (i, 0)),
        pl.BlockSpec((2, _DENSE_BLOCK, HALF), lambda i: (0, i, 0)),
    ],
    out_shape=[
        jax.ShapeDtypeStruct((N_NODES, EMB), jnp.float32),
        jax.ShapeDtypeStruct((2, N_NODES, HALF), jnp.float32),
    ],
)

GCHUNK = BATCH // 32


@functools.partial(
    pl.kernel,
    out_type=[jax.ShapeDtypeStruct((BATCH, 4 * EMB), jnp.float32)
              for _ in range(3)],
    mesh=_mesh,
    scratch_types=[
        pltpu.VMEM((GCHUNK,), jnp.int32),
        pltpu.VMEM((GCHUNK, EMB), jnp.float32),
        pltpu.SemaphoreType.DMA,
    ],
    compiler_params=pltpu.CompilerParams(use_tc_tiling_on_sc=False),
)
def _sc_lookup(t0, t1, t2, t3, users, pos, neg,
               u_out, p_out, n_out, idx_v, buf, sem):
    c = lax.axis_index("c")
    s = lax.axis_index("s")
    w = s * 2 + c
    r0 = w * GCHUNK
    for idx_hbm, out_hbm, off in ((users, u_out, -1),
                                  (pos, p_out, N_USER - 1),
                                  (neg, n_out, N_USER - 1)):
        pltpu.sync_copy(idx_hbm.at[pl.ds(r0, GCHUNK)], idx_v)
        for k in range(GCHUNK // 16):
            sl = pl.ds(k * 16, 16)
            idx_v[sl] = idx_v[sl] + off
        for k, tbl in enumerate((t0, t1, t2, t3)):
            pltpu.async_copy(tbl.at[idx_v], buf, sem).wait()
            pltpu.sync_copy(buf, out_hbm.at[pl.ds(r0, GCHUNK),
                                            pl.ds(k * EMB, EMB)])


def kernel(user_emb, item_emb, edge_index, edge_vals,
           W1_0, b1_0, W2_0, b2_0, W1_1, b1_1, W2_1, b2_1,
           W1_2, b1_2, W2_2, b2_2,
           users, pos_items, neg_items, node_flag):
    Ws = [(W1_0, b1_0, W2_0, b2_0), (W1_1, b1_1, W2_1, b2_1),
          (W1_2, b1_2, W2_2, b2_2)]
    E0 = jnp.concatenate([user_emb, item_emb], axis=0)
    estack = jnp.stack([E0[:, :HALF], E0[:, HALF:]], axis=0)
    src = edge_index[0]
    dst = edge_index[1]
    zeros = jnp.zeros((N_NODES, HALF), jnp.float32)

    norms = []
    for (W1, b1, W2, b2) in Ws:
        lflat = _sc_spmm(estack.reshape(2 * N_NODES, HALF),
                         src, dst, edge_vals, zeros)
        enorm, estack = _dense_tc(lflat.reshape(2, N_NODES, HALF), estack,
                                  W1, b1.reshape(1, EMB),
                                  W2, b2.reshape(1, EMB))
        norms.append(enorm)

    u, p, n = _sc_lookup(E0, norms[0], norms[1], norms[2],
                         users, pos_items, neg_items)
    return (u, p, n)


# speedup vs baseline: 9.3768x; 1.0952x over previous
import functools

import jax
import jax.numpy as jnp
from jax import lax
from jax.experimental import pallas as pl
from jax.experimental.pallas import tpu as pltpu
from jax.experimental.pallas import tpu_sc as plsc

N_USER = 25000
N_ITEM = 25000
N_NODES = N_USER + N_ITEM
EMB = 64
HALF = 32
N_EDGES = 800000
BATCH = 4096

GROUP = 128
N_GROUPS = N_EDGES // GROUP
BASE_GROUPS = N_GROUPS // 16
EXTRA_TILES = N_GROUPS % 16
CH_GROUPS = 13
CH_EDGES = CH_GROUPS * GROUP
N_CHUNKS = BASE_GROUPS // CH_GROUPS
RB_ROWS = 3128
RB_LAST = N_NODES - 15 * RB_ROWS
ZROWS = 125

_mesh = plsc.VectorSubcoreMesh(core_axis_name="c", subcore_axis_name="s")


NSLOT = 3


@functools.partial(
    pl.kernel,
    out_type=jax.ShapeDtypeStruct((2 * N_NODES, HALF), jnp.float32),
    mesh=_mesh,
    scratch_types=[
        pltpu.VMEM((2, CH_EDGES), jnp.int32),
        pltpu.VMEM((2, CH_EDGES), jnp.int32),
        pltpu.VMEM((2, CH_EDGES), jnp.float32),
        pltpu.VMEM((NSLOT, GROUP), jnp.int32),
        pltpu.VMEM((NSLOT, GROUP, HALF), jnp.float32),
        pltpu.VMEM_SHARED((N_NODES, HALF), jnp.float32),
        pltpu.SemaphoreType.DMA((NSLOT,)),
        pltpu.SemaphoreType.DMA((NSLOT,)),
        pltpu.SemaphoreType.DMA,
    ],
    compiler_params=pltpu.CompilerParams(use_tc_tiling_on_sc=False),
)
def _sc_spmm(e_hbm, src_hbm, dst_hbm, vals_hbm, zeros_hbm, out_hbm,
             src_m, dst_m, vals_m, dst_g, rows_v, acc, gsem, ssem, stsem):
    c = lax.axis_index("c")
    s = lax.axis_index("s")
    coff = c * N_NODES

    zr = s * (N_NODES // 16)
    pltpu.sync_copy(zeros_hbm.at[pl.ds(zr, N_NODES // 16)],
                    acc.at[pl.ds(zr, N_NODES // 16)])
    plsc.subcore_barrier()

    g0 = s * BASE_GROUPS + jnp.minimum(s, EXTRA_TILES)
    splat_idx = [jnp.full((16, 1), jj, jnp.int32) for jj in range(16)]
    gdn = lax.GatherDimensionNumbers(offset_dims=(), collapsed_slice_dims=(0,),
                                     start_index_map=(0,))

    def scale_group(pb, b, base):
        def sbody(g, carry):
            vv = vals_m[pb, pl.ds(base + g * 16, 16)]
            for jj in range(16):
                e = g * 16 + jj
                vsp = lax.gather(vv, splat_idx[jj], gdn, (1,),
                                 mode=lax.GatherScatterMode.PROMISE_IN_BOUNDS)
                lo = pl.ds(0, 16)
                hi = pl.ds(16, 16)
                rows_v[b, e, lo] = rows_v[b, e, lo] * vsp
                rows_v[b, e, hi] = rows_v[b, e, hi] * vsp
            return carry

        lax.fori_loop(0, GROUP // 16, sbody, 0)

    def copy_dst(pb, j, b):
        for k in range(GROUP // 16):
            dst_g[b, pl.ds(k * 16, 16)] = dst_m[pb, pl.ds(j * GROUP + k * 16, 16)]

    def start_gather(pb, j, b):
        return pltpu.async_copy(
            e_hbm.at[src_m.at[pb, pl.ds(j * GROUP, GROUP)]],
            rows_v.at[b], gsem.at[b])

    def start_scatter(b):
        return pltpu.async_copy(rows_v.at[b], acc.at[dst_g.at[b]],
                                ssem.at[b], add=True)

    def stage(i, pb):
        eb = (g0 + i * CH_GROUPS) * GROUP
        return (pltpu.make_async_copy(src_hbm.at[pl.ds(eb, CH_EDGES)],
                                      src_m.at[pb], stsem),
                pltpu.make_async_copy(dst_hbm.at[pl.ds(eb, CH_EDGES)],
                                      dst_m.at[pb], stsem),
                pltpu.make_async_copy(vals_hbm.at[pl.ds(eb, CH_EDGES)],
                                      vals_m.at[pb], stsem))

    for cp in stage(0, 0):
        cp.start()

    def chunk_body(i, carry):
        pb = lax.rem(i, 2)
        for cp in stage(i, pb):
            cp.wait()

        @pl.when(i < N_CHUNKS - 1)
        def _prefetch():
            for cp in stage(i + 1, 1 - pb):
                cp.start()

        def offs(k, carry2):
            sl = pl.ds(k * 16, 16)
            src_m[pb, sl] = src_m[pb, sl] + coff
            return carry2

        lax.fori_loop(0, CH_EDGES // 16, offs, 0)

        scat = {}
        prev = None
        for j in range(CH_GROUPS):
            b = j % NSLOT
            if j >= NSLOT:
                scat.pop(j - NSLOT).wait()
            copy_dst(pb, j, b)
            gat = start_gather(pb, j, b)
            if prev is not None:
                pj, pbuf, pgat = prev
                pgat.wait()
                scale_group(pb, pbuf, pj * GROUP)
                scat[pj] = start_scatter(pbuf)
            prev = (j, b, gat)
        pj, pbuf, pgat = prev
        pgat.wait()
        scale_group(pb, pbuf, pj * GROUP)
        scat[pj] = start_scatter(pbuf)
        for j in sorted(scat):
            scat.pop(j).wait()
        return carry

    lax.fori_loop(0, N_CHUNKS, chunk_body, 0)

    @pl.when(s < EXTRA_TILES)
    def _tail():
        eb = (g0 + BASE_GROUPS) * GROUP
        pltpu.sync_copy(src_hbm.at[pl.ds(eb, GROUP)],
                        src_m.at[0, pl.ds(0, GROUP)])
        pltpu.sync_copy(dst_hbm.at[pl.ds(eb, GROUP)],
                        dst_m.at[0, pl.ds(0, GROUP)])
        pltpu.sync_copy(vals_hbm.at[pl.ds(eb, GROUP)],
                        vals_m.at[0, pl.ds(0, GROUP)])

        def offs(k, carry2):
            sl = pl.ds(k * 16, 16)
            src_m[0, sl] = src_m[0, sl] + coff
            return carry2

        lax.fori_loop(0, GROUP // 16, offs, 0)
        copy_dst(0, 0, 0)
        start_gather(0, 0, 0).wait()
        scale_group(0, 0, 0)
        start_scatter(0).wait()

    plsc.subcore_barrier()

    @pl.when(s < 15)
    def _rb_main():
        rb = s * RB_ROWS
        pltpu.sync_copy(acc.at[pl.ds(rb, RB_ROWS)],
                        out_hbm.at[pl.ds(coff + rb, RB_ROWS)])

    @pl.when(s == 15)
    def _rb_last():
        rb = 15 * RB_ROWS
        pltpu.sync_copy(acc.at[pl.ds(rb, RB_LAST)],
                        out_hbm.at[pl.ds(coff + rb, RB_LAST)])


_DENSE_BLOCK = 512
_PB = _DENSE_BLOCK // 4
_DGRID = (N_NODES // 4 + _PB - 1) // _PB


def _dense_body(l_ref, e_ref, w1_ref, w2_ref, gg_ref, p0_ref, p1_ref, b_ref,
                enorm_ref, eo_ref):
    X = jnp.concatenate([l_ref[0] + e_ref[0], l_ref[1] + e_ref[1]], axis=1)
    M = jnp.concatenate([l_ref[0] * e_ref[0], l_ref[1] * e_ref[1]], axis=1)
    H = (jnp.dot(X, w1_ref[...], preferred_element_type=jnp.float32)
         + jnp.dot(M, w2_ref[...], preferred_element_type=jnp.float32)
         + b_ref[0])
    Eo = jnp.where(H >= 0, H, 0.2 * H)
    n2 = jnp.dot(Eo * Eo, gg_ref[...], preferred_element_type=jnp.float32)
    En = Eo / jnp.maximum(jnp.sqrt(n2), 1e-12)
    enorm_ref[...] = En.reshape(2 * _PB, 128)
    eo_ref[0] = jnp.dot(Eo, p0_ref[...], preferred_element_type=jnp.float32)
    eo_ref[1] = jnp.dot(Eo, p1_ref[...], preferred_element_type=jnp.float32)


_dense_tc = pl.pallas_call(
    _dense_body,
    grid=(_DGRID,),
    in_specs=[
        pl.BlockSpec((2, _PB, 128), lambda i: (0, i, 0)),
        pl.BlockSpec((2, _PB, 128), lambda i: (0, i, 0)),
        pl.BlockSpec((256, 256), lambda i: (0, 0)),
        pl.BlockSpec((256, 256), lambda i: (0, 0)),
        pl.BlockSpec((256, 256), lambda i: (0, 0)),
        pl.BlockSpec((256, 128), lambda i: (0, 0)),
        pl.BlockSpec((256, 128), lambda i: (0, 0)),
        pl.BlockSpec((1, 256), lambda i: (0, 0)),
    ],
    out_specs=[
        pl.BlockSpec((2 * _PB, 128), lambda i: (i, 0)),
        pl.BlockSpec((2, _PB, 128), lambda i: (0, i, 0)),
    ],
    out_shape=[
        jax.ShapeDtypeStruct((N_NODES // 2, 128), jnp.float32),
        jax.ShapeDtypeStruct((2, N_NODES // 4, 128), jnp.float32),
    ],
)

GCHUNK = BATCH // 32


@functools.partial(
    pl.kernel,
    out_type=[jax.ShapeDtypeStruct((BATCH, 4 * EMB), jnp.float32)
              for _ in range(3)],
    mesh=_mesh,
    scratch_types=[
        pltpu.VMEM((GCHUNK,), jnp.int32),
        pltpu.VMEM((GCHUNK, EMB), jnp.float32),
        pltpu.SemaphoreType.DMA,
    ],
    compiler_params=pltpu.CompilerParams(use_tc_tiling_on_sc=False),
)
def _sc_lookup(t0, t1, t2, t3, users, pos, neg,
               u_out, p_out, n_out, idx_v, buf, sem):
    c = lax.axis_index("c")
    s = lax.axis_index("s")
    w = s * 2 + c
    r0 = w * GCHUNK
    for idx_hbm, out_hbm, off in ((users, u_out, -1),
                                  (pos, p_out, N_USER - 1),
                                  (neg, n_out, N_USER - 1)):
        pltpu.sync_copy(idx_hbm.at[pl.ds(r0, GCHUNK)], idx_v)
        for k in range(GCHUNK // 16):
            sl = pl.ds(k * 16, 16)
            idx_v[sl] = idx_v[sl] + off
        for k, tbl in enumerate((t0, t1, t2, t3)):
            pltpu.async_copy(tbl.at[idx_v], buf, sem).wait()
            pltpu.sync_copy(buf, out_hbm.at[pl.ds(r0, GCHUNK),
                                            pl.ds(k * EMB, EMB)])


def kernel(user_emb, item_emb, edge_index, edge_vals,
           W1_0, b1_0, W2_0, b2_0, W1_1, b1_1, W2_1, b2_1,
           W1_2, b1_2, W2_2, b2_2,
           users, pos_items, neg_items, node_flag):
    Ws = [(W1_0, b1_0, W2_0, b2_0), (W1_1, b1_1, W2_1, b2_1),
          (W1_2, b1_2, W2_2, b2_2)]
    E0 = jnp.concatenate([user_emb, item_emb], axis=0)
    estack = jnp.stack([E0[:, :HALF].reshape(N_NODES // 4, 128),
                        E0[:, HALF:].reshape(N_NODES // 4, 128)], axis=0)
    src = edge_index[0]
    dst = edge_index[1]
    zeros = jnp.zeros((N_NODES, HALF), jnp.float32)

    e4 = jnp.eye(4, dtype=jnp.float32)
    GG = jnp.kron(e4, jnp.ones((64, 64), jnp.float32))
    P0 = jnp.kron(e4, jnp.eye(64, 32, dtype=jnp.float32))
    P1 = jnp.kron(e4, jnp.eye(64, 32, k=-32, dtype=jnp.float32))

    norms = []
    for (W1, b1, W2, b2) in Ws:
        W1p = jnp.concatenate([jnp.kron(e4, W1[:HALF]),
                               jnp.kron(e4, W1[HALF:])], axis=0)
        W2p = jnp.concatenate([jnp.kron(e4, W2[:HALF]),
                               jnp.kron(e4, W2[HALF:])], axis=0)
        bp = jnp.tile(b1 + b2, 4).reshape(1, 256)
        lflat = _sc_spmm(estack.reshape(2 * N_NODES, HALF),
                         src, dst, edge_vals, zeros)
        enorm, estack = _dense_tc(lflat.reshape(2, N_NODES // 4, 128), estack,
                                  W1p, W2p, GG, P0, P1, bp)
        norms.append(enorm.reshape(N_NODES, EMB))

    u, p, n = _sc_lookup(E0, norms[0], norms[1], norms[2],
                         users, pos_items, neg_items)
    return (u, p, n)


# speedup vs baseline: 10.4421x; 1.1136x over previous
import functools

import jax
import jax.numpy as jnp
from jax import lax
from jax.experimental import pallas as pl
from jax.experimental.pallas import tpu as pltpu
from jax.experimental.pallas import tpu_sc as plsc

N_USER = 25000
N_ITEM = 25000
N_NODES = N_USER + N_ITEM
EMB = 64
HALF = 32
N_EDGES = 800000
BATCH = 4096

GROUP = 128
N_GROUPS = N_EDGES // GROUP
BASE_GROUPS = N_GROUPS // 16
EXTRA_TILES = N_GROUPS % 16
CH_GROUPS = 13
CH_EDGES = CH_GROUPS * GROUP
N_CHUNKS = BASE_GROUPS // CH_GROUPS
RB_ROWS = 3128
RB_LAST = N_NODES - 15 * RB_ROWS
ZROWS = 125

_mesh = plsc.VectorSubcoreMesh(core_axis_name="c", subcore_axis_name="s")


NSLOT = 3


@functools.partial(
    pl.kernel,
    out_type=jax.ShapeDtypeStruct((2 * N_NODES, HALF), jnp.float32),
    mesh=_mesh,
    scratch_types=[
        pltpu.VMEM((2, CH_EDGES), jnp.int32),
        pltpu.VMEM((2, CH_EDGES), jnp.int32),
        pltpu.VMEM((2, CH_EDGES), jnp.float32),
        pltpu.VMEM((NSLOT, GROUP), jnp.int32),
        pltpu.VMEM((NSLOT, GROUP, HALF), jnp.float32),
        pltpu.VMEM_SHARED((N_NODES, HALF), jnp.float32),
        pltpu.SemaphoreType.DMA((NSLOT,)),
        pltpu.SemaphoreType.DMA((NSLOT,)),
        pltpu.SemaphoreType.DMA,
    ],
    compiler_params=pltpu.CompilerParams(use_tc_tiling_on_sc=False),
)
def _sc_spmm(e_hbm, src_hbm, dst_hbm, vals_hbm, zeros_hbm, out_hbm,
             src_m, dst_m, vals_m, dst_g, rows_v, acc, gsem, ssem, stsem):
    c = lax.axis_index("c")
    s = lax.axis_index("s")
    coff = c * N_NODES

    zr = s * (N_NODES // 16)
    pltpu.sync_copy(zeros_hbm.at[pl.ds(zr, N_NODES // 16)],
                    acc.at[pl.ds(zr, N_NODES // 16)])
    plsc.subcore_barrier()

    g0 = s * BASE_GROUPS + jnp.minimum(s, EXTRA_TILES)
    splat_idx = [jnp.full((16, 1), jj, jnp.int32) for jj in range(16)]
    gdn = lax.GatherDimensionNumbers(offset_dims=(), collapsed_slice_dims=(0,),
                                     start_index_map=(0,))

    def scale_group(pb, b, base):
        def sbody(g, carry):
            vv = vals_m[pb, pl.ds(base + g * 16, 16)]
            for jj in range(16):
                e = g * 16 + jj
                vsp = lax.gather(vv, splat_idx[jj], gdn, (1,),
                                 mode=lax.GatherScatterMode.PROMISE_IN_BOUNDS)
                lo = pl.ds(0, 16)
                hi = pl.ds(16, 16)
                rows_v[b, e, lo] = rows_v[b, e, lo] * vsp
                rows_v[b, e, hi] = rows_v[b, e, hi] * vsp
            return carry

        lax.fori_loop(0, GROUP // 16, sbody, 0)

    def copy_dst(pb, j, b):
        for k in range(GROUP // 16):
            dst_g[b, pl.ds(k * 16, 16)] = dst_m[pb, pl.ds(j * GROUP + k * 16, 16)]

    def start_gather(pb, j, b):
        return pltpu.async_copy(
            e_hbm.at[src_m.at[pb, pl.ds(j * GROUP, GROUP)]],
            rows_v.at[b], gsem.at[b])

    def start_scatter(b):
        return pltpu.async_copy(rows_v.at[b], acc.at[dst_g.at[b]],
                                ssem.at[b], add=True)

    def stage(i, pb):
        eb = (g0 + i * CH_GROUPS) * GROUP
        return (pltpu.make_async_copy(src_hbm.at[pl.ds(eb, CH_EDGES)],
                                      src_m.at[pb], stsem),
                pltpu.make_async_copy(dst_hbm.at[pl.ds(eb, CH_EDGES)],
                                      dst_m.at[pb], stsem),
                pltpu.make_async_copy(vals_hbm.at[pl.ds(eb, CH_EDGES)],
                                      vals_m.at[pb], stsem))

    for cp in stage(0, 0):
        cp.start()

    def chunk_body(i, carry):
        pb = lax.rem(i, 2)
        for cp in stage(i, pb):
            cp.wait()

        @pl.when(i < N_CHUNKS - 1)
        def _prefetch():
            for cp in stage(i + 1, 1 - pb):
                cp.start()

        def offs(k, carry2):
            sl = pl.ds(k * 16, 16)
            src_m[pb, sl] = src_m[pb, sl] + coff
            return carry2

        lax.fori_loop(0, CH_EDGES // 16, offs, 0)

        scat = {}
        prev = None
        for j in range(CH_GROUPS):
            b = j % NSLOT
            if j >= NSLOT:
                scat.pop(j - NSLOT).wait()
            copy_dst(pb, j, b)
            gat = start_gather(pb, j, b)
            if prev is not None:
                pj, pbuf, pgat = prev
                pgat.wait()
                scale_group(pb, pbuf, pj * GROUP)
                scat[pj] = start_scatter(pbuf)
            prev = (j, b, gat)
        pj, pbuf, pgat = prev
        pgat.wait()
        scale_group(pb, pbuf, pj * GROUP)
        scat[pj] = start_scatter(pbuf)
        for j in sorted(scat):
            scat.pop(j).wait()
        return carry

    lax.fori_loop(0, N_CHUNKS, chunk_body, 0)

    @pl.when(s < EXTRA_TILES)
    def _tail():
        eb = (g0 + BASE_GROUPS) * GROUP
        pltpu.sync_copy(src_hbm.at[pl.ds(eb, GROUP)],
                        src_m.at[0, pl.ds(0, GROUP)])
        pltpu.sync_copy(dst_hbm.at[pl.ds(eb, GROUP)],
                        dst_m.at[0, pl.ds(0, GROUP)])
        pltpu.sync_copy(vals_hbm.at[pl.ds(eb, GROUP)],
                        vals_m.at[0, pl.ds(0, GROUP)])

        def offs(k, carry2):
            sl = pl.ds(k * 16, 16)
            src_m[0, sl] = src_m[0, sl] + coff
            return carry2

        lax.fori_loop(0, GROUP // 16, offs, 0)
        copy_dst(0, 0, 0)
        start_gather(0, 0, 0).wait()
        scale_group(0, 0, 0)
        start_scatter(0).wait()

    plsc.subcore_barrier()

    @pl.when(s < 15)
    def _rb_main():
        rb = s * RB_ROWS
        pltpu.sync_copy(acc.at[pl.ds(rb, RB_ROWS)],
                        out_hbm.at[pl.ds(coff + rb, RB_ROWS)])

    @pl.when(s == 15)
    def _rb_last():
        rb = 15 * RB_ROWS
        pltpu.sync_copy(acc.at[pl.ds(rb, RB_LAST)],
                        out_hbm.at[pl.ds(coff + rb, RB_LAST)])


_DENSE_BLOCK = 2048
_PB = _DENSE_BLOCK // 4
_DGRID = (N_NODES // 4 + _PB - 1) // _PB


def _dense_body(l_ref, e_ref, w1_ref, w2_ref, gg_ref, p0_ref, p1_ref, b_ref,
                enorm_ref, eo_ref):
    X = jnp.concatenate([l_ref[0] + e_ref[0], l_ref[1] + e_ref[1]], axis=1)
    M = jnp.concatenate([l_ref[0] * e_ref[0], l_ref[1] * e_ref[1]], axis=1)
    H = (jnp.dot(X, w1_ref[...], preferred_element_type=jnp.float32)
         + jnp.dot(M, w2_ref[...], preferred_element_type=jnp.float32)
         + b_ref[0])
    Eo = jnp.where(H >= 0, H, 0.2 * H)
    n2 = jnp.dot(Eo * Eo, gg_ref[...], preferred_element_type=jnp.float32)
    En = Eo / jnp.maximum(jnp.sqrt(n2), 1e-12)
    enorm_ref[...] = En.reshape(2 * _PB, 128)
    eo_ref[0] = jnp.dot(Eo, p0_ref[...], preferred_element_type=jnp.float32)
    eo_ref[1] = jnp.dot(Eo, p1_ref[...], preferred_element_type=jnp.float32)


_dense_tc = pl.pallas_call(
    _dense_body,
    grid=(_DGRID,),
    in_specs=[
        pl.BlockSpec((2, _PB, 128), lambda i: (0, i, 0)),
        pl.BlockSpec((2, _PB, 128), lambda i: (0, i, 0)),
        pl.BlockSpec((256, 256), lambda i: (0, 0)),
        pl.BlockSpec((256, 256), lambda i: (0, 0)),
        pl.BlockSpec((256, 256), lambda i: (0, 0)),
        pl.BlockSpec((256, 128), lambda i: (0, 0)),
        pl.BlockSpec((256, 128), lambda i: (0, 0)),
        pl.BlockSpec((1, 256), lambda i: (0, 0)),
    ],
    out_specs=[
        pl.BlockSpec((2 * _PB, 128), lambda i: (i, 0)),
        pl.BlockSpec((2, _PB, 128), lambda i: (0, i, 0)),
    ],
    out_shape=[
        jax.ShapeDtypeStruct((N_NODES // 2, 128), jnp.float32),
        jax.ShapeDtypeStruct((2, N_NODES // 4, 128), jnp.float32),
    ],
)

GCHUNK = BATCH // 32


@functools.partial(
    pl.kernel,
    out_type=[jax.ShapeDtypeStruct((BATCH, 4 * EMB), jnp.float32)
              for _ in range(3)],
    mesh=_mesh,
    scratch_types=[
        pltpu.VMEM((GCHUNK,), jnp.int32),
        pltpu.VMEM((GCHUNK, EMB), jnp.float32),
        pltpu.SemaphoreType.DMA,
    ],
    compiler_params=pltpu.CompilerParams(use_tc_tiling_on_sc=False),
)
def _sc_lookup(t0, t1, t2, t3, users, pos, neg,
               u_out, p_out, n_out, idx_v, buf, sem):
    c = lax.axis_index("c")
    s = lax.axis_index("s")
    w = s * 2 + c
    r0 = w * GCHUNK
    for idx_hbm, out_hbm, off in ((users, u_out, -1),
                                  (pos, p_out, N_USER - 1),
                                  (neg, n_out, N_USER - 1)):
        pltpu.sync_copy(idx_hbm.at[pl.ds(r0, GCHUNK)], idx_v)
        for k in range(GCHUNK // 16):
            sl = pl.ds(k * 16, 16)
            idx_v[sl] = idx_v[sl] + off
        for k, tbl in enumerate((t0, t1, t2, t3)):
            pltpu.async_copy(tbl.at[idx_v], buf, sem).wait()
            pltpu.sync_copy(buf, out_hbm.at[pl.ds(r0, GCHUNK),
                                            pl.ds(k * EMB, EMB)])


def kernel(user_emb, item_emb, edge_index, edge_vals,
           W1_0, b1_0, W2_0, b2_0, W1_1, b1_1, W2_1, b2_1,
           W1_2, b1_2, W2_2, b2_2,
           users, pos_items, neg_items, node_flag):
    Ws = [(W1_0, b1_0, W2_0, b2_0), (W1_1, b1_1, W2_1, b2_1),
          (W1_2, b1_2, W2_2, b2_2)]
    E0 = jnp.concatenate([user_emb, item_emb], axis=0)
    estack = jnp.stack([E0[:, :HALF].reshape(N_NODES // 4, 128),
                        E0[:, HALF:].reshape(N_NODES // 4, 128)], axis=0)
    src = edge_index[0]
    dst = edge_index[1]
    zeros = jnp.zeros((N_NODES, HALF), jnp.float32)

    e4 = jnp.eye(4, dtype=jnp.float32)
    GG = jnp.kron(e4, jnp.ones((64, 64), jnp.float32))
    P0 = jnp.kron(e4, jnp.eye(64, 32, dtype=jnp.float32))
    P1 = jnp.kron(e4, jnp.eye(64, 32, k=-32, dtype=jnp.float32))

    norms = []
    for (W1, b1, W2, b2) in Ws:
        W1p = jnp.concatenate([jnp.kron(e4, W1[:HALF]),
                               jnp.kron(e4, W1[HALF:])], axis=0)
        W2p = jnp.concatenate([jnp.kron(e4, W2[:HALF]),
                               jnp.kron(e4, W2[HALF:])], axis=0)
        bp = jnp.tile(b1 + b2, 4).reshape(1, 256)
        lflat = _sc_spmm(estack.reshape(2 * N_NODES, HALF),
                         src, dst, edge_vals, zeros)
        enorm, estack = _dense_tc(lflat.reshape(2, N_NODES // 4, 128), estack,
                                  W1p, W2p, GG, P0, P1, bp)
        norms.append(enorm.reshape(N_NODES, EMB))

    u, p, n = _sc_lookup(E0, norms[0], norms[1], norms[2],
                         users, pos_items, neg_items)
    return (u, p, n)


# speedup vs baseline: 10.6504x; 1.0200x over previous
import functools

import jax
import jax.numpy as jnp
from jax import lax
from jax.experimental import pallas as pl
from jax.experimental.pallas import tpu as pltpu
from jax.experimental.pallas import tpu_sc as plsc

N_USER = 25000
N_ITEM = 25000
N_NODES = N_USER + N_ITEM
EMB = 64
HALF = 32
N_EDGES = 800000
BATCH = 4096

GROUP = 128
N_GROUPS = N_EDGES // GROUP
BASE_GROUPS = N_GROUPS // 16
EXTRA_TILES = N_GROUPS % 16
CH_GROUPS = 13
CH_EDGES = CH_GROUPS * GROUP
N_CHUNKS = BASE_GROUPS // CH_GROUPS
RB_ROWS = 3128
RB_LAST = N_NODES - 15 * RB_ROWS
ZROWS = 125

_mesh = plsc.VectorSubcoreMesh(core_axis_name="c", subcore_axis_name="s")


NSLOT = 3


@functools.partial(
    pl.kernel,
    out_type=jax.ShapeDtypeStruct((2 * N_NODES, HALF), jnp.float32),
    mesh=_mesh,
    scratch_types=[
        pltpu.VMEM((2, CH_EDGES), jnp.int32),
        pltpu.VMEM((2, CH_EDGES), jnp.int32),
        pltpu.VMEM((2, CH_EDGES), jnp.float32),
        pltpu.VMEM((NSLOT, GROUP), jnp.int32),
        pltpu.VMEM((NSLOT, GROUP, HALF), jnp.float32),
        pltpu.VMEM_SHARED((N_NODES, HALF), jnp.float32),
        pltpu.SemaphoreType.DMA((NSLOT,)),
        pltpu.SemaphoreType.DMA((NSLOT,)),
        pltpu.SemaphoreType.DMA,
    ],
    compiler_params=pltpu.CompilerParams(use_tc_tiling_on_sc=False),
)
def _sc_spmm(e_hbm, src_hbm, dst_hbm, vals_hbm, zeros_hbm, out_hbm,
             src_m, dst_m, vals_m, dst_g, rows_v, acc, gsem, ssem, stsem):
    c = lax.axis_index("c")
    s = lax.axis_index("s")
    coff = c * N_NODES

    zr = s * (N_NODES // 16)
    pltpu.sync_copy(zeros_hbm.at[pl.ds(zr, N_NODES // 16)],
                    acc.at[pl.ds(zr, N_NODES // 16)])
    plsc.subcore_barrier()

    g0 = s * BASE_GROUPS + jnp.minimum(s, EXTRA_TILES)
    splat_idx = [jnp.full((16, 1), jj, jnp.int32) for jj in range(16)]
    gdn = lax.GatherDimensionNumbers(offset_dims=(), collapsed_slice_dims=(0,),
                                     start_index_map=(0,))

    def scale_group(pb, b, base):
        def sbody(g, carry):
            vv = vals_m[pb, pl.ds(base + g * 16, 16)]
            for jj in range(16):
                e = g * 16 + jj
                vsp = lax.gather(vv, splat_idx[jj], gdn, (1,),
                                 mode=lax.GatherScatterMode.PROMISE_IN_BOUNDS)
                lo = pl.ds(0, 16)
                hi = pl.ds(16, 16)
                rows_v[b, e, lo] = rows_v[b, e, lo] * vsp
                rows_v[b, e, hi] = rows_v[b, e, hi] * vsp
            return carry

        lax.fori_loop(0, GROUP // 16, sbody, 0)

    def copy_dst(pb, j, b):
        for k in range(GROUP // 16):
            dst_g[b, pl.ds(k * 16, 16)] = dst_m[pb, pl.ds(j * GROUP + k * 16, 16)]

    def start_gather(pb, j, b):
        return pltpu.async_copy(
            e_hbm.at[src_m.at[pb, pl.ds(j * GROUP, GROUP)]],
            rows_v.at[b], gsem.at[b])

    def start_scatter(b):
        return pltpu.async_copy(rows_v.at[b], acc.at[dst_g.at[b]],
                                ssem.at[b], add=True)

    def stage(i, pb):
        eb = (g0 + i * CH_GROUPS) * GROUP
        return (pltpu.make_async_copy(src_hbm.at[pl.ds(eb, CH_EDGES)],
                                      src_m.at[pb], stsem),
                pltpu.make_async_copy(dst_hbm.at[pl.ds(eb, CH_EDGES)],
                                      dst_m.at[pb], stsem),
                pltpu.make_async_copy(vals_hbm.at[pl.ds(eb, CH_EDGES)],
                                      vals_m.at[pb], stsem))

    for cp in stage(0, 0):
        cp.start()

    def chunk_body(i, carry):
        pb = lax.rem(i, 2)
        for cp in stage(i, pb):
            cp.wait()

        @pl.when(i < N_CHUNKS - 1)
        def _prefetch():
            for cp in stage(i + 1, 1 - pb):
                cp.start()

        def offs(k, carry2):
            sl = pl.ds(k * 16, 16)
            src_m[pb, sl] = src_m[pb, sl] + coff
            return carry2

        lax.fori_loop(0, CH_EDGES // 16, offs, 0)

        scat = {}
        prev = None
        for j in range(CH_GROUPS):
            b = j % NSLOT
            if j >= NSLOT:
                scat.pop(j - NSLOT).wait()
            copy_dst(pb, j, b)
            gat = start_gather(pb, j, b)
            if prev is not None:
                pj, pbuf, pgat = prev
                pgat.wait()
                scale_group(pb, pbuf, pj * GROUP)
                scat[pj] = start_scatter(pbuf)
            prev = (j, b, gat)
        pj, pbuf, pgat = prev
        pgat.wait()
        scale_group(pb, pbuf, pj * GROUP)
        scat[pj] = start_scatter(pbuf)
        for j in sorted(scat):
            scat.pop(j).wait()
        return carry

    lax.fori_loop(0, N_CHUNKS, chunk_body, 0)

    @pl.when(s < EXTRA_TILES)
    def _tail():
        eb = (g0 + BASE_GROUPS) * GROUP
        pltpu.sync_copy(src_hbm.at[pl.ds(eb, GROUP)],
                        src_m.at[0, pl.ds(0, GROUP)])
        pltpu.sync_copy(dst_hbm.at[pl.ds(eb, GROUP)],
                        dst_m.at[0, pl.ds(0, GROUP)])
        pltpu.sync_copy(vals_hbm.at[pl.ds(eb, GROUP)],
                        vals_m.at[0, pl.ds(0, GROUP)])

        def offs(k, carry2):
            sl = pl.ds(k * 16, 16)
            src_m[0, sl] = src_m[0, sl] + coff
            return carry2

        lax.fori_loop(0, GROUP // 16, offs, 0)
        copy_dst(0, 0, 0)
        start_gather(0, 0, 0).wait()
        scale_group(0, 0, 0)
        start_scatter(0).wait()

    plsc.subcore_barrier()

    @pl.when(s < 15)
    def _rb_main():
        rb = s * RB_ROWS
        pltpu.sync_copy(acc.at[pl.ds(rb, RB_ROWS)],
                        out_hbm.at[pl.ds(coff + rb, RB_ROWS)])

    @pl.when(s == 15)
    def _rb_last():
        rb = 15 * RB_ROWS
        pltpu.sync_copy(acc.at[pl.ds(rb, RB_LAST)],
                        out_hbm.at[pl.ds(coff + rb, RB_LAST)])


_DENSE_BLOCK = 4096
_PB = _DENSE_BLOCK // 4
_DGRID = (N_NODES // 4 + _PB - 1) // _PB


def _dense_body(l_ref, e_ref, w1_ref, w2_ref, gg_ref, p0_ref, p1_ref, b_ref,
                enorm_ref, eo_ref):
    X = jnp.concatenate([l_ref[0] + e_ref[0], l_ref[1] + e_ref[1]], axis=1)
    M = jnp.concatenate([l_ref[0] * e_ref[0], l_ref[1] * e_ref[1]], axis=1)
    H = (jnp.dot(X, w1_ref[...], preferred_element_type=jnp.float32)
         + jnp.dot(M, w2_ref[...], preferred_element_type=jnp.float32)
         + b_ref[0])
    Eo = jnp.where(H >= 0, H, 0.2 * H)
    n2 = jnp.dot(Eo * Eo, gg_ref[...], preferred_element_type=jnp.float32)
    En = Eo / jnp.maximum(jnp.sqrt(n2), 1e-12)
    enorm_ref[...] = En.reshape(2 * _PB, 128)
    eo_ref[0] = jnp.dot(Eo, p0_ref[...], preferred_element_type=jnp.float32)
    eo_ref[1] = jnp.dot(Eo, p1_ref[...], preferred_element_type=jnp.float32)


_dense_tc = pl.pallas_call(
    _dense_body,
    grid=(_DGRID,),
    in_specs=[
        pl.BlockSpec((2, _PB, 128), lambda i: (0, i, 0)),
        pl.BlockSpec((2, _PB, 128), lambda i: (0, i, 0)),
        pl.BlockSpec((256, 256), lambda i: (0, 0)),
        pl.BlockSpec((256, 256), lambda i: (0, 0)),
        pl.BlockSpec((256, 256), lambda i: (0, 0)),
        pl.BlockSpec((256, 128), lambda i: (0, 0)),
        pl.BlockSpec((256, 128), lambda i: (0, 0)),
        pl.BlockSpec((1, 256), lambda i: (0, 0)),
    ],
    out_specs=[
        pl.BlockSpec((2 * _PB, 128), lambda i: (i, 0)),
        pl.BlockSpec((2, _PB, 128), lambda i: (0, i, 0)),
    ],
    out_shape=[
        jax.ShapeDtypeStruct((N_NODES // 2, 128), jnp.float32),
        jax.ShapeDtypeStruct((2, N_NODES // 4, 128), jnp.float32),
    ],
)

GCHUNK = BATCH // 32


@functools.partial(
    pl.kernel,
    out_type=[jax.ShapeDtypeStruct((BATCH, 4 * EMB), jnp.float32)
              for _ in range(3)],
    mesh=_mesh,
    scratch_types=[
        pltpu.VMEM((GCHUNK,), jnp.int32),
        pltpu.VMEM((GCHUNK, EMB), jnp.float32),
        pltpu.SemaphoreType.DMA,
    ],
    compiler_params=pltpu.CompilerParams(use_tc_tiling_on_sc=False),
)
def _sc_lookup(t0, t1, t2, t3, users, pos, neg,
               u_out, p_out, n_out, idx_v, buf, sem):
    c = lax.axis_index("c")
    s = lax.axis_index("s")
    w = s * 2 + c
    r0 = w * GCHUNK
    for idx_hbm, out_hbm, off in ((users, u_out, -1),
                                  (pos, p_out, N_USER - 1),
                                  (neg, n_out, N_USER - 1)):
        pltpu.sync_copy(idx_hbm.at[pl.ds(r0, GCHUNK)], idx_v)
        for k in range(GCHUNK // 16):
            sl = pl.ds(k * 16, 16)
            idx_v[sl] = idx_v[sl] + off
        for k, tbl in enumerate((t0, t1, t2, t3)):
            pltpu.async_copy(tbl.at[idx_v], buf, sem).wait()
            pltpu.sync_copy(buf, out_hbm.at[pl.ds(r0, GCHUNK),
                                            pl.ds(k * EMB, EMB)])


def kernel(user_emb, item_emb, edge_index, edge_vals,
           W1_0, b1_0, W2_0, b2_0, W1_1, b1_1, W2_1, b2_1,
           W1_2, b1_2, W2_2, b2_2,
           users, pos_items, neg_items, node_flag):
    Ws = [(W1_0, b1_0, W2_0, b2_0), (W1_1, b1_1, W2_1, b2_1),
          (W1_2, b1_2, W2_2, b2_2)]
    E0 = jnp.concatenate([user_emb, item_emb], axis=0)
    estack = jnp.stack([E0[:, :HALF].reshape(N_NODES // 4, 128),
                        E0[:, HALF:].reshape(N_NODES // 4, 128)], axis=0)
    src = edge_index[0]
    dst = edge_index[1]
    zeros = jnp.zeros((N_NODES, HALF), jnp.float32)

    e4 = jnp.eye(4, dtype=jnp.float32)
    GG = jnp.kron(e4, jnp.ones((64, 64), jnp.float32))
    P0 = jnp.kron(e4, jnp.eye(64, 32, dtype=jnp.float32))
    P1 = jnp.kron(e4, jnp.eye(64, 32, k=-32, dtype=jnp.float32))

    norms = []
    for (W1, b1, W2, b2) in Ws:
        W1p = jnp.concatenate([jnp.kron(e4, W1[:HALF]),
                               jnp.kron(e4, W1[HALF:])], axis=0)
        W2p = jnp.concatenate([jnp.kron(e4, W2[:HALF]),
                               jnp.kron(e4, W2[HALF:])], axis=0)
        bp = jnp.tile(b1 + b2, 4).reshape(1, 256)
        lflat = _sc_spmm(estack.reshape(2 * N_NODES, HALF),
                         src, dst, edge_vals, zeros)
        enorm, estack = _dense_tc(lflat.reshape(2, N_NODES // 4, 128), estack,
                                  W1p, W2p, GG, P0, P1, bp)
        norms.append(enorm.reshape(N_NODES, EMB))

    u, p, n = _sc_lookup(E0, norms[0], norms[1], norms[2],
                         users, pos_items, neg_items)
    return (u, p, n)


# speedup vs baseline: 10.8592x; 1.0196x over previous
import functools

import jax
import jax.numpy as jnp
from jax import lax
from jax.experimental import pallas as pl
from jax.experimental.pallas import tpu as pltpu
from jax.experimental.pallas import tpu_sc as plsc

N_USER = 25000
N_ITEM = 25000
N_NODES = N_USER + N_ITEM
EMB = 64
HALF = 32
N_EDGES = 800000
BATCH = 4096

GROUP = 128
N_GROUPS = N_EDGES // GROUP
BASE_GROUPS = N_GROUPS // 16
EXTRA_TILES = N_GROUPS % 16
CH_GROUPS = 13
CH_EDGES = CH_GROUPS * GROUP
N_CHUNKS = BASE_GROUPS // CH_GROUPS
RB_ROWS = 3128
RB_LAST = N_NODES - 15 * RB_ROWS
ZROWS = 125

_mesh = plsc.VectorSubcoreMesh(core_axis_name="c", subcore_axis_name="s")


NSLOT = 3


@functools.partial(
    pl.kernel,
    out_type=jax.ShapeDtypeStruct((2 * N_NODES, HALF), jnp.float32),
    mesh=_mesh,
    scratch_types=[
        pltpu.VMEM((2, CH_EDGES), jnp.int32),
        pltpu.VMEM((2, CH_EDGES), jnp.int32),
        pltpu.VMEM((2, CH_EDGES), jnp.float32),
        pltpu.VMEM((NSLOT, GROUP), jnp.int32),
        pltpu.VMEM((NSLOT, GROUP, HALF), jnp.float32),
        pltpu.VMEM((ZROWS, HALF), jnp.float32),
        pltpu.VMEM_SHARED((N_NODES, HALF), jnp.float32),
        pltpu.SemaphoreType.DMA((NSLOT,)),
        pltpu.SemaphoreType.DMA((NSLOT,)),
        pltpu.SemaphoreType.DMA,
        pltpu.SemaphoreType.DMA,
    ],
    compiler_params=pltpu.CompilerParams(use_tc_tiling_on_sc=False),
)
def _sc_spmm(e_hbm, src_hbm, dst_hbm, vals_hbm, out_hbm,
             src_m, dst_m, vals_m, dst_g, rows_v, zbuf, acc,
             gsem, ssem, stsem, zsem):
    c = lax.axis_index("c")
    s = lax.axis_index("s")
    coff = c * N_NODES

    g0 = s * BASE_GROUPS + jnp.minimum(s, EXTRA_TILES)
    splat_idx = [jnp.full((16, 1), jj, jnp.int32) for jj in range(16)]
    gdn = lax.GatherDimensionNumbers(offset_dims=(), collapsed_slice_dims=(0,),
                                     start_index_map=(0,))

    def scale_group(pb, b, base):
        def sbody(g, carry):
            vv = vals_m[pb, pl.ds(base + g * 16, 16)]
            for jj in range(16):
                e = g * 16 + jj
                vsp = lax.gather(vv, splat_idx[jj], gdn, (1,),
                                 mode=lax.GatherScatterMode.PROMISE_IN_BOUNDS)
                lo = pl.ds(0, 16)
                hi = pl.ds(16, 16)
                rows_v[b, e, lo] = rows_v[b, e, lo] * vsp
                rows_v[b, e, hi] = rows_v[b, e, hi] * vsp
            return carry

        lax.fori_loop(0, GROUP // 16, sbody, 0)

    def copy_dst(pb, j, b):
        for k in range(GROUP // 16):
            dst_g[b, pl.ds(k * 16, 16)] = dst_m[pb, pl.ds(j * GROUP + k * 16, 16)]

    def start_gather(pb, j, b):
        return pltpu.async_copy(
            e_hbm.at[src_m.at[pb, pl.ds(j * GROUP, GROUP)]],
            rows_v.at[b], gsem.at[b])

    def start_scatter(b):
        return pltpu.async_copy(rows_v.at[b], acc.at[dst_g.at[b]],
                                ssem.at[b], add=True)

    def stage(i, pb):
        eb = (g0 + i * CH_GROUPS) * GROUP
        return (pltpu.make_async_copy(src_hbm.at[pl.ds(eb, CH_EDGES)],
                                      src_m.at[pb], stsem),
                pltpu.make_async_copy(dst_hbm.at[pl.ds(eb, CH_EDGES)],
                                      dst_m.at[pb], stsem),
                pltpu.make_async_copy(vals_hbm.at[pl.ds(eb, CH_EDGES)],
                                      vals_m.at[pb], stsem))

    for cp in stage(0, 0):
        cp.start()

    z16 = jnp.zeros((16,), jnp.float32)

    def zrow(r, carry):
        zbuf[r, pl.ds(0, 16)] = z16
        zbuf[r, pl.ds(16, 16)] = z16
        return carry

    lax.fori_loop(0, ZROWS, zrow, 0)
    zr = s * (N_NODES // 16)
    zcps = [pltpu.make_async_copy(
        zbuf, acc.at[pl.ds(zr + i * ZROWS, ZROWS)], zsem)
        for i in range((N_NODES // 16) // ZROWS)]
    for cp in zcps:
        cp.start()
    for cp in zcps:
        cp.wait()
    plsc.subcore_barrier()

    def chunk_body(i, carry):
        pb = lax.rem(i, 2)
        for cp in stage(i, pb):
            cp.wait()

        @pl.when(i < N_CHUNKS - 1)
        def _prefetch():
            for cp in stage(i + 1, 1 - pb):
                cp.start()

        def offs(k, carry2):
            sl = pl.ds(k * 16, 16)
            src_m[pb, sl] = src_m[pb, sl] + coff
            return carry2

        lax.fori_loop(0, CH_EDGES // 16, offs, 0)

        scat = {}
        prev = None
        for j in range(CH_GROUPS):
            b = j % NSLOT
            if j >= NSLOT:
                scat.pop(j - NSLOT).wait()
            copy_dst(pb, j, b)
            gat = start_gather(pb, j, b)
            if prev is not None:
                pj, pbuf, pgat = prev
                pgat.wait()
                scale_group(pb, pbuf, pj * GROUP)
                scat[pj] = start_scatter(pbuf)
            prev = (j, b, gat)
        pj, pbuf, pgat = prev
        pgat.wait()
        scale_group(pb, pbuf, pj * GROUP)
        scat[pj] = start_scatter(pbuf)
        for j in sorted(scat):
            scat.pop(j).wait()
        return carry

    lax.fori_loop(0, N_CHUNKS, chunk_body, 0)

    @pl.when(s < EXTRA_TILES)
    def _tail():
        eb = (g0 + BASE_GROUPS) * GROUP
        pltpu.sync_copy(src_hbm.at[pl.ds(eb, GROUP)],
                        src_m.at[0, pl.ds(0, GROUP)])
        pltpu.sync_copy(dst_hbm.at[pl.ds(eb, GROUP)],
                        dst_m.at[0, pl.ds(0, GROUP)])
        pltpu.sync_copy(vals_hbm.at[pl.ds(eb, GROUP)],
                        vals_m.at[0, pl.ds(0, GROUP)])

        def offs(k, carry2):
            sl = pl.ds(k * 16, 16)
            src_m[0, sl] = src_m[0, sl] + coff
            return carry2

        lax.fori_loop(0, GROUP // 16, offs, 0)
        copy_dst(0, 0, 0)
        start_gather(0, 0, 0).wait()
        scale_group(0, 0, 0)
        start_scatter(0).wait()

    plsc.subcore_barrier()

    @pl.when(s < 15)
    def _rb_main():
        rb = s * RB_ROWS
        pltpu.sync_copy(acc.at[pl.ds(rb, RB_ROWS)],
                        out_hbm.at[pl.ds(coff + rb, RB_ROWS)])

    @pl.when(s == 15)
    def _rb_last():
        rb = 15 * RB_ROWS
        pltpu.sync_copy(acc.at[pl.ds(rb, RB_LAST)],
                        out_hbm.at[pl.ds(coff + rb, RB_LAST)])


_DENSE_BLOCK = 4096
_PB = _DENSE_BLOCK // 4
_DGRID = (N_NODES // 4 + _PB - 1) // _PB


def _dense_body(l_ref, e_ref, w1_ref, w2_ref, gg_ref, p0_ref, p1_ref, b_ref,
                enorm_ref, eo_ref):
    X = jnp.concatenate([l_ref[0] + e_ref[0], l_ref[1] + e_ref[1]], axis=1)
    M = jnp.concatenate([l_ref[0] * e_ref[0], l_ref[1] * e_ref[1]], axis=1)
    H = (jnp.dot(X, w1_ref[...], preferred_element_type=jnp.float32)
         + jnp.dot(M, w2_ref[...], preferred_element_type=jnp.float32)
         + b_ref[0])
    Eo = jnp.where(H >= 0, H, 0.2 * H)
    n2 = jnp.dot(Eo * Eo, gg_ref[...], preferred_element_type=jnp.float32)
    En = Eo / jnp.maximum(jnp.sqrt(n2), 1e-12)
    enorm_ref[...] = En.reshape(2 * _PB, 128)
    eo_ref[0] = jnp.dot(Eo, p0_ref[...], preferred_element_type=jnp.float32)
    eo_ref[1] = jnp.dot(Eo, p1_ref[...], preferred_element_type=jnp.float32)


_dense_tc = pl.pallas_call(
    _dense_body,
    grid=(_DGRID,),
    in_specs=[
        pl.BlockSpec((2, _PB, 128), lambda i: (0, i, 0)),
        pl.BlockSpec((2, _PB, 128), lambda i: (0, i, 0)),
        pl.BlockSpec((256, 256), lambda i: (0, 0)),
        pl.BlockSpec((256, 256), lambda i: (0, 0)),
        pl.BlockSpec((256, 256), lambda i: (0, 0)),
        pl.BlockSpec((256, 128), lambda i: (0, 0)),
        pl.BlockSpec((256, 128), lambda i: (0, 0)),
        pl.BlockSpec((1, 256), lambda i: (0, 0)),
    ],
    out_specs=[
        pl.BlockSpec((2 * _PB, 128), lambda i: (i, 0)),
        pl.BlockSpec((2, _PB, 128), lambda i: (0, i, 0)),
    ],
    out_shape=[
        jax.ShapeDtypeStruct((N_NODES // 2, 128), jnp.float32),
        jax.ShapeDtypeStruct((2, N_NODES // 4, 128), jnp.float32),
    ],
)

GCHUNK = BATCH // 32


@functools.partial(
    pl.kernel,
    out_type=[jax.ShapeDtypeStruct((BATCH, 4 * EMB), jnp.float32)
              for _ in range(3)],
    mesh=_mesh,
    scratch_types=[
        pltpu.VMEM((GCHUNK,), jnp.int32),
        pltpu.VMEM((GCHUNK, EMB), jnp.float32),
        pltpu.SemaphoreType.DMA,
    ],
    compiler_params=pltpu.CompilerParams(use_tc_tiling_on_sc=False),
)
def _sc_lookup(t0, t1, t2, t3, users, pos, neg,
               u_out, p_out, n_out, idx_v, buf, sem):
    c = lax.axis_index("c")
    s = lax.axis_index("s")
    w = s * 2 + c
    r0 = w * GCHUNK
    for idx_hbm, out_hbm, off in ((users, u_out, -1),
                                  (pos, p_out, N_USER - 1),
                                  (neg, n_out, N_USER - 1)):
        pltpu.sync_copy(idx_hbm.at[pl.ds(r0, GCHUNK)], idx_v)
        for k in range(GCHUNK // 16):
            sl = pl.ds(k * 16, 16)
            idx_v[sl] = idx_v[sl] + off
        for k, tbl in enumerate((t0, t1, t2, t3)):
            pltpu.async_copy(tbl.at[idx_v], buf, sem).wait()
            pltpu.sync_copy(buf, out_hbm.at[pl.ds(r0, GCHUNK),
                                            pl.ds(k * EMB, EMB)])


def kernel(user_emb, item_emb, edge_index, edge_vals,
           W1_0, b1_0, W2_0, b2_0, W1_1, b1_1, W2_1, b2_1,
           W1_2, b1_2, W2_2, b2_2,
           users, pos_items, neg_items, node_flag):
    Ws = [(W1_0, b1_0, W2_0, b2_0), (W1_1, b1_1, W2_1, b2_1),
          (W1_2, b1_2, W2_2, b2_2)]
    E0 = jnp.concatenate([user_emb, item_emb], axis=0)
    estack = jnp.stack([E0[:, :HALF].reshape(N_NODES // 4, 128),
                        E0[:, HALF:].reshape(N_NODES // 4, 128)], axis=0)
    src = edge_index[0]
    dst = edge_index[1]

    e4 = jnp.eye(4, dtype=jnp.float32)
    GG = jnp.kron(e4, jnp.ones((64, 64), jnp.float32))
    P0 = jnp.kron(e4, jnp.eye(64, 32, dtype=jnp.float32))
    P1 = jnp.kron(e4, jnp.eye(64, 32, k=-32, dtype=jnp.float32))

    norms = []
    for (W1, b1, W2, b2) in Ws:
        W1p = jnp.concatenate([jnp.kron(e4, W1[:HALF]),
                               jnp.kron(e4, W1[HALF:])], axis=0)
        W2p = jnp.concatenate([jnp.kron(e4, W2[:HALF]),
                               jnp.kron(e4, W2[HALF:])], axis=0)
        bp = jnp.tile(b1 + b2, 4).reshape(1, 256)
        lflat = _sc_spmm(estack.reshape(2 * N_NODES, HALF),
                         src, dst, edge_vals)
        enorm, estack = _dense_tc(lflat.reshape(2, N_NODES // 4, 128), estack,
                                  W1p, W2p, GG, P0, P1, bp)
        norms.append(enorm.reshape(N_NODES, EMB))

    u, p, n = _sc_lookup(E0, norms[0], norms[1], norms[2],
                         users, pos_items, neg_items)
    return (u, p, n)
